# baseline (speedup 1.0000x reference)
import functools

import jax
import jax.numpy as jnp
from jax import lax
from jax.experimental import pallas as pl
from jax.experimental.pallas import tpu as pltpu
from jax.experimental.pallas import tpu_sc as plsc

CB = 8192
D = 256
BETA = 0.25

N = 4 * 4 * 32 * 32
BN = 256
NB = N // BN
GRP = 8 * 342
LOSS_SCALE = (1.0 + BETA) / (N * D)


def _vq_body(a_ref, en_ref, flat_ref, e_ref, idx_ref, loss_ref):
    i = pl.program_id(0)
    f = flat_ref[...]
    e = e_ref[...]
    aa = a_ref[...]
    en = en_ref[...]
    mm = lax.dot_general(f, e, (((1,), (1,)), ((), ())),
                         preferred_element_type=jnp.float32,
                         precision=lax.Precision.DEFAULT)
    dist = (aa + en) - 2.0 * mm
    col = lax.broadcasted_iota(jnp.int32, dist.shape, 1)

    state_v = None
    for lo, hi in ((0, GRP), (GRP, 2 * GRP), (2 * GRP, CB)):
        m = (col >= lo) & (col < hi)
        dg = jnp.where(m, dist, jnp.inf)
        m_g = jnp.min(dg, axis=1, keepdims=True)
        i_g = jnp.min(jnp.where(dg == m_g, col, CB), axis=1, keepdims=True)
        m_g_bf = m_g.astype(jnp.bfloat16).astype(jnp.float32)
        if state_v is None:
            state_v, idx, true_v = m_g_bf, i_g, m_g
        else:
            upd = m_g < state_v
            idx = jnp.where(upd, i_g, idx)
            true_v = jnp.where(upd, m_g, true_v)
            state_v = jnp.where(upd, m_g_bf, state_v)
    idx_ref[...] = idx

    part = jnp.sum(true_v, axis=(0, 1), keepdims=True)
    prev = jnp.where(i == 0, jnp.zeros_like(part), loss_ref[...])
    tot = prev + part
    loss_ref[...] = jnp.where(i == NB - 1, tot * LOSS_SCALE, tot)


@functools.cache
def _get_argmin_call():
    return pl.pallas_call(
        _vq_body,
        grid=(NB,),
        in_specs=[
            pl.BlockSpec((BN, 1), lambda i: (i, 0)),
            pl.BlockSpec((1, CB), lambda i: (0, 0)),
            pl.BlockSpec((BN, D), lambda i: (i, 0)),
            pl.BlockSpec((CB, D), lambda i: (0, 0)),
        ],
        out_specs=[
            pl.BlockSpec((BN, 1), lambda i: (i, 0)),
            pl.BlockSpec((1, 1), lambda i: (0, 0)),
        ],
        out_shape=[
            jax.ShapeDtypeStruct((N, 1), jnp.int32),
            jax.ShapeDtypeStruct((1, 1), jnp.float32),
        ],
    )


_NC, _NS = 2, 16
_NW = _NC * _NS
_BPW = N // _NW
_CH = 128


def _sc_gather_body(e_hbm, idx_hbm, out_hbm, idx_v, rows_v, sem):
    wid = lax.axis_index("s") * _NC + lax.axis_index("c")
    base = wid * _BPW
    for c in range(_BPW // _CH):
        row0 = base + c * _CH
        pltpu.sync_copy(idx_hbm.at[pl.ds(row0, _CH)], idx_v)
        pltpu.async_copy(e_hbm.at[idx_v], rows_v, sem).wait()
        pltpu.sync_copy(rows_v, out_hbm.at[pl.ds(row0, _CH)])


@functools.cache
def _get_sc_gather():
    return pl.kernel(
        _sc_gather_body,
        out_type=jax.ShapeDtypeStruct((N, D), jnp.float32),
        mesh=plsc.VectorSubcoreMesh(core_axis_name="c", subcore_axis_name="s"),
        scratch_types=[
            pltpu.VMEM((_CH,), jnp.int32),
            pltpu.VMEM((_CH, D), jnp.float32),
            pltpu.SemaphoreType.DMA,
        ],
    )


def kernel(z, E):
    B, C, T, H, W = z.shape
    zp = jnp.transpose(z, (0, 2, 3, 4, 1))
    latents_shape = zp.shape
    flat = zp.reshape(-1, D)
    a = jnp.sum(zp ** 2, axis=4).reshape(-1, 1)
    en = jnp.sum(E ** 2, axis=1).reshape(1, CB)

    idx2d, loss = _get_argmin_call()(a, en, flat, E)
    vq_loss = loss[0, 0]

    zq_flat = _get_sc_gather()(E, idx2d.reshape(-1))

    z_q = zq_flat.reshape(latents_shape)
    out = jnp.transpose(z_q, (0, 4, 1, 2, 3))
    return (out, vq_loss, idx2d, latents_shape)
